# baseline (speedup 1.0000x reference)
import jax
import jax.numpy as jnp
from jax.experimental import pallas as pl
from jax.experimental.pallas import tpu as pltpu

NUM_CLASSES = 80
PB = 1984
LG = 128


def _assign_kernel(pts_ref, rr_ref, gr_ref, st_ref, gt_ref, lab_ref,
                   cls_ref, reg_ref):
    f32 = jnp.float32
    gb = gt_ref[...]
    gx = gb[0:1, :]
    gy = gb[1:2, :]
    gw = gb[2:3, :]
    gh = gb[3:4, :]
    gxe = gx + gw - 1.0
    gye = gy + gh - 1.0
    cx = gx + gw / 2.0
    cy = gy + gh / 2.0
    meas = jnp.maximum(gw, gh)
    lab_row = lab_ref[0].astype(f32)

    half = jax.lax.broadcasted_iota(jnp.int32, (LG, 1), 0) >= LG // 2
    m5 = jnp.concatenate([gx, gy, gxe, gye, lab_row], axis=0).T
    z3 = jnp.zeros((LG, 3), f32)
    gm = jnp.concatenate(
        [jnp.where(half, 0.0, m5), z3, jnp.where(half, m5, 0.0), z3],
        axis=1)
    lab_col = lab_ref[0].T + jnp.where(half, 128, 0)
    oh = (jax.lax.broadcasted_iota(jnp.int32, (LG, 256), 1) ==
          lab_col).astype(f32)

    pts = pts_ref[...]
    px = pts[:, 0:1]
    py = pts[:, 1:2]

    d1 = px - gx
    d2 = py - gy
    d3 = gxe - px
    d4 = gye - py
    hit = jnp.minimum(jnp.minimum(d1, d2), jnp.minimum(d3, d4)) >= 0.0
    hf = hit.astype(f32)
    f1 = d1 * hf
    f2 = d2 * hf
    f3 = d3 * hf
    f4 = d4 * hf
    q = (jnp.maximum(jnp.minimum(f1, f3), 0.0) /
         jnp.maximum(jnp.maximum(f1, f3), 0.01)) * \
        (jnp.maximum(jnp.minimum(f2, f4), 0.0) /
         jnp.maximum(jnp.maximum(f2, f4), 0.01))
    q = jnp.sqrt(q)

    s2 = st_ref[...] * 0.5
    inside_core = (px >= cx - s2) & (px <= cx + s2) & \
                  (py >= cy - s2) & (py <= cy + s2) & hit
    q = jnp.where(inside_core, 1.0, q)

    rr = rr_ref[...]
    gr = gr_ref[...]
    rlo = rr[:, 0:1]
    rhi = rr[:, 1:2]
    glo = gr[:, 0:1]
    ghi = gr[:, 1:2]
    green = (rlo <= meas) & (meas <= rhi) & hit
    gray = (((glo <= meas) & (meas < rlo)) |
            ((rhi < meas) & (meas <= ghi))) & hit
    q = q * green.astype(f32)

    PBn = q.shape[0]
    G = LG // 2
    qA = q[:, 0:G]
    qB = q[:, G:LG]
    maxA = jnp.max(qA, axis=1, keepdims=True)
    maxB = jnp.max(qB, axis=1, keepdims=True)
    maxfull = jnp.concatenate(
        [jnp.broadcast_to(maxA, (PBn, G)), jnp.broadcast_to(maxB, (PBn, G))],
        axis=1)
    gidx = jnp.bitwise_and(
        jax.lax.broadcasted_iota(jnp.int32, (PBn, LG), 1), G - 1)
    cand = jnp.where(q == maxfull, gidx, G)
    midxA = jnp.min(cand[:, 0:G], axis=1, keepdims=True)
    midxB = jnp.min(cand[:, G:LG], axis=1, keepdims=True)
    midxfull = jnp.concatenate(
        [jnp.broadcast_to(midxA, (PBn, G)), jnp.broadcast_to(midxB, (PBn, G))],
        axis=1)
    sel = (gidx == midxfull).astype(f32)

    t = jnp.dot(sel, gm, preferred_element_type=f32,
                precision=jax.lax.Precision.HIGHEST)
    gcv = jnp.dot(gray.astype(f32), oh, preferred_element_type=f32)

    ciota = jax.lax.broadcasted_iota(jnp.int32, (PBn, NUM_CLASSES), 1)

    posA = maxA > 0.0
    valA = jnp.where(posA, maxA, 0.0)
    matchedA = t[:, 4:5].astype(jnp.int32)
    clsA = jnp.where(ciota == matchedA, valA, 0.0)
    grayA = gcv[:, 0:NUM_CLASSES] > 0.0
    clsA = jnp.where(grayA & (clsA == 0.0), -1.0, clsA)
    cls_ref[0] = clsA
    regA = jnp.concatenate(
        [px - t[:, 0:1], py - t[:, 1:2], t[:, 2:3] - px, t[:, 3:4] - py],
        axis=1) * posA.astype(f32)
    reg_ref[0] = regA

    posB = maxB > 0.0
    valB = jnp.where(posB, maxB, 0.0)
    matchedB = t[:, 12:13].astype(jnp.int32)
    clsB = jnp.where(ciota == matchedB, valB, 0.0)
    grayB = gcv[:, 128:128 + NUM_CLASSES] > 0.0
    clsB = jnp.where(grayB & (clsB == 0.0), -1.0, clsB)
    cls_ref[1] = clsB
    regB = jnp.concatenate(
        [px - t[:, 8:9], py - t[:, 9:10], t[:, 10:11] - px, t[:, 11:12] - py],
        axis=1) * posB.astype(f32)
    reg_ref[1] = regB


def kernel(points, reg_ranges, gray_ranges, strides, gt_bboxes, gt_labels):
    P = points.shape[0]
    B, G, _ = gt_bboxes.shape
    assert P % PB == 0 and B % 2 == 0 and 2 * G == LG
    npb = P // PB
    B2 = B // 2

    strides2 = strides.reshape(P, 1)
    gt_t = gt_bboxes.reshape(B * G, 4).T
    lab2 = gt_labels.reshape(B2, 1, LG)

    cls, reg = pl.pallas_call(
        _assign_kernel,
        grid=(B2, npb),
        in_specs=[
            pl.BlockSpec((PB, 2), lambda b, i: (i, 0)),
            pl.BlockSpec((PB, 2), lambda b, i: (i, 0)),
            pl.BlockSpec((PB, 2), lambda b, i: (i, 0)),
            pl.BlockSpec((PB, 1), lambda b, i: (i, 0)),
            pl.BlockSpec((4, LG), lambda b, i: (0, b)),
            pl.BlockSpec((1, 1, LG), lambda b, i: (b, 0, 0)),
        ],
        out_specs=[
            pl.BlockSpec((2, PB, NUM_CLASSES), lambda b, i: (b, i, 0)),
            pl.BlockSpec((2, PB, 4), lambda b, i: (b, i, 0)),
        ],
        out_shape=[
            jax.ShapeDtypeStruct((B, P, NUM_CLASSES), jnp.float32),
            jax.ShapeDtypeStruct((B, P, 4), jnp.float32),
        ],
        compiler_params=pltpu.CompilerParams(
            dimension_semantics=("parallel", "arbitrary"),
        ),
    )(points, reg_ranges, gray_ranges, strides2, gt_t, lab2)
    return cls, reg


# speedup vs baseline: 1.0817x; 1.0817x over previous
import jax
import jax.numpy as jnp
from jax.experimental import pallas as pl
from jax.experimental.pallas import tpu as pltpu

NUM_CLASSES = 80
PB = 1984
LG = 128


def _assign_kernel(pt_ref, gt_ref, lab_ref, cls_ref, reg_ref):
    f32 = jnp.float32
    gb = gt_ref[...]
    gx = gb[0:1, :]
    gy = gb[1:2, :]
    gw = gb[2:3, :]
    gh = gb[3:4, :]
    gxe = gx + gw - 1.0
    gye = gy + gh - 1.0
    cx = gx + gw / 2.0
    cy = gy + gh / 2.0
    meas = jnp.maximum(gw, gh)
    lab_row = lab_ref[0].astype(f32)

    half = jax.lax.broadcasted_iota(jnp.int32, (LG, 1), 0) >= LG // 2
    m5 = jnp.concatenate([gx, gy, gxe, gye, lab_row], axis=0).T
    z3 = jnp.zeros((LG, 3), f32)
    gm = jnp.concatenate(
        [jnp.where(half, 0.0, m5), z3, jnp.where(half, m5, 0.0), z3],
        axis=1)
    lab_col = lab_ref[0].T + jnp.where(half, 128, 0)
    oh = (jax.lax.broadcasted_iota(jnp.int32, (LG, 256), 1) ==
          lab_col).astype(f32)

    pts = pt_ref[...]
    px = pts[:, 0:1]
    py = pts[:, 1:2]

    d1 = px - gx
    d2 = py - gy
    d3 = gxe - px
    d4 = gye - py
    hit = jnp.minimum(jnp.minimum(d1, d2), jnp.minimum(d3, d4)) >= 0.0
    hf = hit.astype(f32)
    f1 = d1 * hf
    f2 = d2 * hf
    f3 = d3 * hf
    f4 = d4 * hf
    q = (jnp.maximum(jnp.minimum(f1, f3), 0.0) /
         jnp.maximum(jnp.maximum(f1, f3), 0.01)) * \
        (jnp.maximum(jnp.minimum(f2, f4), 0.0) /
         jnp.maximum(jnp.maximum(f2, f4), 0.01))
    q = jnp.sqrt(q)

    s2 = pts[:, 6:7] * 0.5
    inside_core = (px >= cx - s2) & (px <= cx + s2) & \
                  (py >= cy - s2) & (py <= cy + s2) & hit
    q = jnp.where(inside_core, 1.0, q)

    rlo = pts[:, 2:3]
    rhi = pts[:, 3:4]
    glo = pts[:, 4:5]
    ghi = pts[:, 5:6]
    green = (rlo <= meas) & (meas <= rhi) & hit
    gray = (((glo <= meas) & (meas < rlo)) |
            ((rhi < meas) & (meas <= ghi))) & hit
    q = q * green.astype(f32)

    PBn = q.shape[0]
    G = LG // 2
    qA = q[:, 0:G]
    qB = q[:, G:LG]
    maxA = jnp.max(qA, axis=1, keepdims=True)
    maxB = jnp.max(qB, axis=1, keepdims=True)
    maxfull = jnp.concatenate(
        [jnp.broadcast_to(maxA, (PBn, G)), jnp.broadcast_to(maxB, (PBn, G))],
        axis=1)
    gidx = jnp.bitwise_and(
        jax.lax.broadcasted_iota(jnp.int32, (PBn, LG), 1), G - 1)
    cand = jnp.where(q == maxfull, gidx, G)
    midxA = jnp.min(cand[:, 0:G], axis=1, keepdims=True)
    midxB = jnp.min(cand[:, G:LG], axis=1, keepdims=True)
    midxfull = jnp.concatenate(
        [jnp.broadcast_to(midxA, (PBn, G)), jnp.broadcast_to(midxB, (PBn, G))],
        axis=1)
    sel = (gidx == midxfull).astype(f32)

    t = jnp.dot(sel, gm, preferred_element_type=f32,
                precision=jax.lax.Precision.HIGHEST)
    gcv = jnp.dot(gray.astype(f32), oh, preferred_element_type=f32)

    ciota = jax.lax.broadcasted_iota(jnp.int32, (PBn, NUM_CLASSES), 1)

    posA = maxA > 0.0
    valA = jnp.where(posA, maxA, 0.0)
    matchedA = t[:, 4:5].astype(jnp.int32)
    clsA = jnp.where(ciota == matchedA, valA, 0.0)
    grayA = gcv[:, 0:NUM_CLASSES] > 0.0
    clsA = jnp.where(grayA & (clsA == 0.0), -1.0, clsA)
    cls_ref[0] = clsA
    regA = jnp.concatenate(
        [px - t[:, 0:1], py - t[:, 1:2], t[:, 2:3] - px, t[:, 3:4] - py],
        axis=1) * posA.astype(f32)
    reg_ref[0] = regA

    posB = maxB > 0.0
    valB = jnp.where(posB, maxB, 0.0)
    matchedB = t[:, 12:13].astype(jnp.int32)
    clsB = jnp.where(ciota == matchedB, valB, 0.0)
    grayB = gcv[:, 128:128 + NUM_CLASSES] > 0.0
    clsB = jnp.where(grayB & (clsB == 0.0), -1.0, clsB)
    cls_ref[1] = clsB
    regB = jnp.concatenate(
        [px - t[:, 8:9], py - t[:, 9:10], t[:, 10:11] - px, t[:, 11:12] - py],
        axis=1) * posB.astype(f32)
    reg_ref[1] = regB


def kernel(points, reg_ranges, gray_ranges, strides, gt_bboxes, gt_labels):
    P = points.shape[0]
    B, G, _ = gt_bboxes.shape
    assert P % PB == 0 and B % 2 == 0 and 2 * G == LG
    npb = P // PB
    B2 = B // 2

    ptall = jnp.concatenate(
        [points, reg_ranges, gray_ranges, strides.reshape(P, 1)], axis=1)
    gt_t = gt_bboxes.reshape(B * G, 4).T
    lab2 = gt_labels.reshape(B2, 1, LG)

    cls, reg = pl.pallas_call(
        _assign_kernel,
        grid=(B2, npb),
        in_specs=[
            pl.BlockSpec((PB, 7), lambda b, i: (i, 0)),
            pl.BlockSpec((4, LG), lambda b, i: (0, b)),
            pl.BlockSpec((1, 1, LG), lambda b, i: (b, 0, 0)),
        ],
        out_specs=[
            pl.BlockSpec((2, PB, NUM_CLASSES), lambda b, i: (b, i, 0)),
            pl.BlockSpec((2, PB, 4), lambda b, i: (b, i, 0)),
        ],
        out_shape=[
            jax.ShapeDtypeStruct((B, P, NUM_CLASSES), jnp.float32),
            jax.ShapeDtypeStruct((B, P, 4), jnp.float32),
        ],
        compiler_params=pltpu.CompilerParams(
            dimension_semantics=("parallel", "arbitrary"),
        ),
    )(ptall, gt_t, lab2)
    return cls, reg


# speedup vs baseline: 2.3695x; 2.1904x over previous
import jax
import jax.numpy as jnp
from jax.experimental import pallas as pl
from jax.experimental.pallas import tpu as pltpu

NUM_CLASSES = 80
NL = 2432
PPAD = 21888
LG = 128


def _assign_kernel(pt_ref, gtr_ref, gtt_ref, labr_ref, cls_ref, reg_ref):
    f32 = jnp.float32
    pt = pt_ref[...]
    px = pt[0:1, :]
    py = pt[1:2, :]
    rlo = pt[2:3, :]
    rhi = pt[3:4, :]
    glo = pt[4:5, :]
    ghi = pt[5:6, :]
    s2 = pt[6:7, :] * 0.5

    gbb = gtr_ref[...]
    gx = gbb[:, 0:1]
    gy = gbb[:, 1:2]
    gw = gbb[:, 2:3]
    gh = gbb[:, 3:4]
    gxe = gx + gw - 1.0
    gye = gy + gh - 1.0
    cx = gx + gw / 2.0
    cy = gy + gh / 2.0
    meas = jnp.maximum(gw, gh)

    d1 = px - gx
    d2 = py - gy
    d3 = gxe - px
    d4 = gye - py
    hit = jnp.minimum(jnp.minimum(d1, d2), jnp.minimum(d3, d4)) >= 0.0
    hf = hit.astype(f32)
    f1 = d1 * hf
    f2 = d2 * hf
    f3 = d3 * hf
    f4 = d4 * hf
    q = (jnp.maximum(jnp.minimum(f1, f3), 0.0) /
         jnp.maximum(jnp.maximum(f1, f3), 0.01)) * \
        (jnp.maximum(jnp.minimum(f2, f4), 0.0) /
         jnp.maximum(jnp.maximum(f2, f4), 0.01))
    q = jnp.sqrt(q)

    inside_core = (px >= cx - s2) & (px <= cx + s2) & \
                  (py >= cy - s2) & (py <= cy + s2) & hit
    q = jnp.where(inside_core, 1.0, q)

    green = (rlo <= meas) & (meas <= rhi) & hit
    gray = (((glo <= meas) & (meas < rlo)) |
            ((rhi < meas) & (meas <= ghi))) & hit
    q = q * green.astype(f32)

    G = LG // 2
    maxA = jnp.max(q[0:G, :], axis=0, keepdims=True)
    maxB = jnp.max(q[G:LG, :], axis=0, keepdims=True)
    sub_iota = jax.lax.broadcasted_iota(jnp.int32, (LG, NL), 0)
    gidx = jnp.bitwise_and(sub_iota, G - 1)
    is_a = sub_iota < G
    maxfull = jnp.where(is_a, maxA, maxB)
    cand = jnp.where(q == maxfull, gidx, G)
    midxA = jnp.min(cand[0:G, :], axis=0, keepdims=True)
    midxB = jnp.min(cand[G:LG, :], axis=0, keepdims=True)
    midxfull = jnp.where(is_a, midxA, midxB)
    sel = (gidx == midxfull).astype(f32)

    gbr = gtt_ref[...]
    gx_r = gbr[0:1, :]
    gy_r = gbr[1:2, :]
    gxe_r = gx_r + gbr[2:3, :] - 1.0
    gye_r = gy_r + gbr[3:4, :] - 1.0
    lab_r = labr_ref[0].astype(f32)
    half_r = jax.lax.broadcasted_iota(jnp.int32, (1, LG), 1) >= G
    m5 = jnp.concatenate([gx_r, gy_r, gxe_r, gye_r, lab_r], axis=0)
    z3 = jnp.zeros((3, LG), f32)
    gm = jnp.concatenate(
        [jnp.where(half_r, 0.0, m5), z3, jnp.where(half_r, m5, 0.0), z3],
        axis=0)
    target_r = labr_ref[0] + jnp.where(half_r, 128, 0)
    oh = (jax.lax.broadcasted_iota(jnp.int32, (256, LG), 0) ==
          target_r).astype(f32)

    t = jnp.dot(gm, sel, preferred_element_type=f32,
                precision=jax.lax.Precision.HIGHEST)
    gcv = jnp.dot(oh, gray.astype(f32), preferred_element_type=f32)

    ciota = jax.lax.broadcasted_iota(jnp.int32, (NUM_CLASSES, NL), 0)

    posA = maxA > 0.0
    valA = jnp.where(posA, maxA, 0.0)
    matchedA = t[4:5, :].astype(jnp.int32)
    clsA = jnp.where(ciota == matchedA, valA, 0.0)
    grayA = gcv[0:NUM_CLASSES, :] > 0.0
    clsA = jnp.where(grayA & (clsA == 0.0), -1.0, clsA)
    cls_ref[0] = clsA
    regA = jnp.concatenate(
        [px - t[0:1, :], py - t[1:2, :], t[2:3, :] - px, t[3:4, :] - py],
        axis=0) * posA.astype(f32)
    reg_ref[0] = regA

    posB = maxB > 0.0
    valB = jnp.where(posB, maxB, 0.0)
    matchedB = t[12:13, :].astype(jnp.int32)
    clsB = jnp.where(ciota == matchedB, valB, 0.0)
    grayB = gcv[128:128 + NUM_CLASSES, :] > 0.0
    clsB = jnp.where(grayB & (clsB == 0.0), -1.0, clsB)
    cls_ref[1] = clsB
    regB = jnp.concatenate(
        [px - t[8:9, :], py - t[9:10, :], t[10:11, :] - px, t[11:12, :] - py],
        axis=0) * posB.astype(f32)
    reg_ref[1] = regB


def kernel(points, reg_ranges, gray_ranges, strides, gt_bboxes, gt_labels):
    P = points.shape[0]
    B, G, _ = gt_bboxes.shape
    assert PPAD % NL == 0 and B % 2 == 0 and 2 * G == LG
    npb = PPAD // NL
    B2 = B // 2

    ptall = jnp.concatenate(
        [points, reg_ranges, gray_ranges, strides.reshape(P, 1)], axis=1)
    pt_t = jnp.pad(ptall, ((0, PPAD - P), (0, 0))).T
    gt_raw = gt_bboxes.reshape(B * G, 4)
    gt_t = gt_bboxes.reshape(B * G, 4).T
    lab_row = gt_labels.reshape(B2, 1, LG)

    cls_t, reg_t = pl.pallas_call(
        _assign_kernel,
        grid=(B2, npb),
        in_specs=[
            pl.BlockSpec((7, NL), lambda b, i: (0, i)),
            pl.BlockSpec((LG, 4), lambda b, i: (b, 0)),
            pl.BlockSpec((4, LG), lambda b, i: (0, b)),
            pl.BlockSpec((1, 1, LG), lambda b, i: (b, 0, 0)),
        ],
        out_specs=[
            pl.BlockSpec((2, NUM_CLASSES, NL), lambda b, i: (b, 0, i)),
            pl.BlockSpec((2, 4, NL), lambda b, i: (b, 0, i)),
        ],
        out_shape=[
            jax.ShapeDtypeStruct((B, NUM_CLASSES, PPAD), jnp.float32),
            jax.ShapeDtypeStruct((B, 4, PPAD), jnp.float32),
        ],
        compiler_params=pltpu.CompilerParams(
            dimension_semantics=("parallel", "arbitrary"),
        ),
    )(pt_t, gt_raw, gt_t, lab_row)
    cls = jnp.transpose(cls_t[:, :, :P], (0, 2, 1))
    reg = jnp.transpose(reg_t[:, :, :P], (0, 2, 1))
    return cls, reg


# speedup vs baseline: 2.5397x; 1.0718x over previous
import jax
import jax.numpy as jnp
from jax.experimental import pallas as pl
from jax.experimental.pallas import tpu as pltpu

NUM_CLASSES = 80
NL = 7296
PPAD = 21888
LG = 128


def _assign_kernel(pt_ref, gtr_ref, gtt_ref, labr_ref, cls_ref, reg_ref):
    f32 = jnp.float32
    pt = pt_ref[...]
    px = pt[0:1, :]
    py = pt[1:2, :]
    rlo = pt[2:3, :]
    rhi = pt[3:4, :]
    glo = pt[4:5, :]
    ghi = pt[5:6, :]
    s2 = pt[6:7, :] * 0.5

    gbb = gtr_ref[...]
    gx = gbb[:, 0:1]
    gy = gbb[:, 1:2]
    gw = gbb[:, 2:3]
    gh = gbb[:, 3:4]
    gxe = gx + gw - 1.0
    gye = gy + gh - 1.0
    cx = gx + gw / 2.0
    cy = gy + gh / 2.0
    meas = jnp.maximum(gw, gh)

    d1 = px - gx
    d2 = py - gy
    d3 = gxe - px
    d4 = gye - py
    hit = jnp.minimum(jnp.minimum(d1, d2), jnp.minimum(d3, d4)) >= 0.0
    q = (jnp.maximum(jnp.minimum(d1, d3), 0.0) /
         jnp.maximum(jnp.maximum(d1, d3), 0.01)) * \
        (jnp.maximum(jnp.minimum(d2, d4), 0.0) /
         jnp.maximum(jnp.maximum(d2, d4), 0.01))
    q = jnp.sqrt(q)

    inside_core = (px >= cx - s2) & (px <= cx + s2) & \
                  (py >= cy - s2) & (py <= cy + s2) & hit
    q = jnp.where(inside_core, 1.0, q)

    ghs = (rlo <= meas) & (meas <= rhi)
    green = ghs & hit
    gray = (glo <= meas) & (meas <= ghi) & ~ghs & hit
    q = q * green.astype(f32)

    G = LG // 2
    qA = q[0:G, :]
    qB = q[G:LG, :]
    maxA = jnp.max(qA, axis=0, keepdims=True)
    maxB = jnp.max(qB, axis=0, keepdims=True)
    gidx = jax.lax.broadcasted_iota(jnp.int32, (G, NL), 0)
    candA = jnp.where(qA == maxA, gidx, G)
    candB = jnp.where(qB == maxB, gidx, G)
    midxA = jnp.min(candA, axis=0, keepdims=True)
    midxB = jnp.min(candB, axis=0, keepdims=True)
    sel = jnp.concatenate(
        [(gidx == midxA), (gidx == midxB)], axis=0).astype(f32)

    gbr = gtt_ref[...]
    gx_r = gbr[0:1, :]
    gy_r = gbr[1:2, :]
    gxe_r = gx_r + gbr[2:3, :] - 1.0
    gye_r = gy_r + gbr[3:4, :] - 1.0
    lab_r = labr_ref[0].astype(f32)
    half_r = jax.lax.broadcasted_iota(jnp.int32, (1, LG), 1) >= G
    m5 = jnp.concatenate([gx_r, gy_r, gxe_r, gye_r, lab_r], axis=0)
    z3 = jnp.zeros((3, LG), f32)
    gm = jnp.concatenate(
        [jnp.where(half_r, 0.0, m5), z3, jnp.where(half_r, m5, 0.0), z3],
        axis=0)
    target_r = labr_ref[0] + jnp.where(half_r, 128, 0)
    oh = (jax.lax.broadcasted_iota(jnp.int32, (256, LG), 0) ==
          target_r).astype(f32)

    t = jnp.dot(gm, sel, preferred_element_type=f32,
                precision=jax.lax.Precision.HIGHEST)
    gcv = jnp.dot(oh, gray.astype(f32), preferred_element_type=f32)

    ciota = jax.lax.broadcasted_iota(jnp.int32, (NUM_CLASSES, NL), 0)

    posA = maxA > 0.0
    valA = jnp.where(posA, maxA, 0.0)
    matchedA = t[4:5, :].astype(jnp.int32)
    clsA = jnp.where(ciota == matchedA, valA, 0.0)
    grayA = gcv[0:NUM_CLASSES, :] > 0.0
    clsA = jnp.where(grayA & (clsA == 0.0), -1.0, clsA)
    cls_ref[0] = clsA
    regA = jnp.concatenate(
        [px - t[0:1, :], py - t[1:2, :], t[2:3, :] - px, t[3:4, :] - py],
        axis=0) * posA.astype(f32)
    reg_ref[0] = regA

    posB = maxB > 0.0
    valB = jnp.where(posB, maxB, 0.0)
    matchedB = t[12:13, :].astype(jnp.int32)
    clsB = jnp.where(ciota == matchedB, valB, 0.0)
    grayB = gcv[128:128 + NUM_CLASSES, :] > 0.0
    clsB = jnp.where(grayB & (clsB == 0.0), -1.0, clsB)
    cls_ref[1] = clsB
    regB = jnp.concatenate(
        [px - t[8:9, :], py - t[9:10, :], t[10:11, :] - px, t[11:12, :] - py],
        axis=0) * posB.astype(f32)
    reg_ref[1] = regB


def kernel(points, reg_ranges, gray_ranges, strides, gt_bboxes, gt_labels):
    P = points.shape[0]
    B, G, _ = gt_bboxes.shape
    assert PPAD % NL == 0 and B % 2 == 0 and 2 * G == LG
    npb = PPAD // NL
    B2 = B // 2

    ptall = jnp.concatenate(
        [points, reg_ranges, gray_ranges, strides.reshape(P, 1)], axis=1)
    pt_t = jnp.pad(ptall, ((0, PPAD - P), (0, 0))).T
    gt_raw = gt_bboxes.reshape(B * G, 4)
    gt_t = gt_bboxes.reshape(B * G, 4).T
    lab_row = gt_labels.reshape(B2, 1, LG)

    cls_t, reg_t = pl.pallas_call(
        _assign_kernel,
        grid=(B2, npb),
        in_specs=[
            pl.BlockSpec((7, NL), lambda b, i: (0, i)),
            pl.BlockSpec((LG, 4), lambda b, i: (b, 0)),
            pl.BlockSpec((4, LG), lambda b, i: (0, b)),
            pl.BlockSpec((1, 1, LG), lambda b, i: (b, 0, 0)),
        ],
        out_specs=[
            pl.BlockSpec((2, NUM_CLASSES, NL), lambda b, i: (b, 0, i)),
            pl.BlockSpec((2, 4, NL), lambda b, i: (b, 0, i)),
        ],
        out_shape=[
            jax.ShapeDtypeStruct((B, NUM_CLASSES, PPAD), jnp.float32),
            jax.ShapeDtypeStruct((B, 4, PPAD), jnp.float32),
        ],
        compiler_params=pltpu.CompilerParams(
            dimension_semantics=("parallel", "arbitrary"),
        ),
    )(pt_t, gt_raw, gt_t, lab_row)
    cls = jnp.transpose(cls_t[:, :, :P], (0, 2, 1))
    reg = jnp.transpose(reg_t[:, :, :P], (0, 2, 1))
    return cls, reg
